# baseline (speedup 1.0000x reference)
import functools

import jax
import jax.numpy as jnp
from jax import lax
from jax.experimental import pallas as pl
from jax.experimental.pallas import tpu as pltpu
from jax.experimental.pallas import tpu_sc as plsc

N = 10000
E = 160000
NC = 2
NS = 16
NP = 10240
FC = 64

AGG_ROWS = 80
DEG_ROWS = 40

BR = 2000
NR = N // BR


@functools.cache
def _mesh():
    return plsc.VectorSubcoreMesh(
        core_axis_name="c", subcore_axis_name="s", num_cores=NC, num_subcores=NS
    )


def _deg_body(dst_ref, zer_ref, one_ref, out_ref, ones_v, idx_v, acc_s):
    c = lax.axis_index("c")
    s = lax.axis_index("s")
    @pl.when(s < 10)
    def _():
        pltpu.sync_copy(zer_ref, acc_s.at[pl.ds(s * 1024, 1024)])

    pltpu.sync_copy(one_ref, ones_v)
    pltpu.sync_copy(dst_ref.at[pl.ds(s * AGG_ROWS + c * DEG_ROWS, DEG_ROWS)],
                    idx_v)
    plsc.subcore_barrier()

    def batch(j, carry):
        pltpu.sync_copy(ones_v, acc_s.at[idx_v.at[j]], add=True)
        return carry

    lax.fori_loop(0, DEG_ROWS, batch, 0)
    plsc.subcore_barrier()

    @pl.when(s < 10)
    def _():
        r0 = s * 1024
        pltpu.sync_copy(acc_s.at[pl.ds(r0, 1024)],
                        out_ref.at[pl.ds(r0, 1024), pl.ds(c * 8, 8)])


@functools.cache
def _deg_kernel():
    return pl.kernel(
        _deg_body,
        out_type=jax.ShapeDtypeStruct((NP, 2 * 8), jnp.float32),
        mesh=_mesh(),
        compiler_params=pltpu.CompilerParams(use_tc_tiling_on_sc=False),
        scratch_types=[
            pltpu.VMEM((128, 8), jnp.float32),
            pltpu.VMEM((DEG_ROWS, 128), jnp.int32),
            pltpu.VMEM_SHARED((NP, 8), jnp.float32),
        ],
    )


def _agg_body(hp_ref, srcq_ref, dst_ref, out_ref,
              sidx_v, didx_v, rows_vs, gsems, ssems, acc_s, *, ph):
    c = lax.axis_index("c")
    s = lax.axis_index("s")
    nb = len(rows_vs)
    pltpu.sync_copy(dst_ref.at[pl.ds(s * AGG_ROWS, AGG_ROWS)], didx_v)

    for p in range(ph):
        q = 2 * p + c
        pltpu.sync_copy(srcq_ref.at[pl.ds((q * NS + s) * AGG_ROWS, AGG_ROWS)],
                        sidx_v)
        @pl.when(s < 10)
        def _():
            pltpu.sync_copy(hp_ref.at[pl.ds(q * N + s * 1000, 1000)],
                            acc_s.at[pl.ds(s * 1000, 1000)])

        plsc.subcore_barrier()

        def gather(m, b):
            return pltpu.make_async_copy(hp_ref.at[sidx_v.at[m]],
                                         rows_vs[b], gsems[b])

        def scatter(m, b):
            return pltpu.make_async_copy(rows_vs[b], acc_s.at[didx_v.at[m]],
                                         ssems[b])

        G = 7
        for b in range(G):
            gather(b, b).start()

        def batch(j, carry):
            m0 = nb * j
            for u in range(nb):
                m = m0 + u
                gather(m, u).wait()
                scatter(m, u).start(add=True)
                k = m + G
                bk = (u + G) % nb

                @pl.when(k < AGG_ROWS)
                def _():
                    @pl.when(k >= nb)
                    def _():
                        scatter(k - nb, bk).wait()

                    gather(k, bk).start()

            return carry

        lax.fori_loop(0, AGG_ROWS // nb, batch, 0)
        for u in range(nb):
            scatter(AGG_ROWS - nb + u, (AGG_ROWS - nb + u) % nb).wait()
        plsc.subcore_barrier()

        @pl.when(s < 10)
        def _():
            pltpu.sync_copy(acc_s.at[pl.ds(s * 1000, 1000)],
                            out_ref.at[pl.ds(q * N + s * 1000, 1000)])

        plsc.subcore_barrier()


@functools.cache
def _agg_kernel(nch):
    ph = nch // NC
    return pl.kernel(
        functools.partial(_agg_body, ph=ph),
        out_type=jax.ShapeDtypeStruct((nch * N, FC), jnp.float32),
        mesh=_mesh(),
        compiler_params=pltpu.CompilerParams(use_tc_tiling_on_sc=False),
        scratch_types=[
            pltpu.VMEM((AGG_ROWS, 128), jnp.int32),
            pltpu.VMEM((AGG_ROWS, 128), jnp.int32),
            tuple(pltpu.VMEM((128, FC), jnp.float32) for _ in range(8)),
            tuple(pltpu.SemaphoreType.DMA for _ in range(8)),
            tuple(pltpu.SemaphoreType.DMA for _ in range(8)),
            pltpu.VMEM_SHARED((NP, FC), jnp.float32),
        ],
    )


def _dis_block(p_ref):
    p = p_ref[...]
    return lax.rsqrt(p[:, 0:1] + p[:, 8:9] + 1.0)


def _mm1_body(x_ref, w_ref, p_ref, hp_ref):
    dis = _dis_block(p_ref)
    h = jnp.dot(x_ref[...], w_ref[...], preferred_element_type=jnp.float32)
    hp_ref[...] = h * dis


def _mid_body(s0_ref, s1_ref, s2_ref, s3_ref, p_ref, b_ref, w_ref, hp_ref):
    dis = _dis_block(p_ref)
    b = b_ref[...]
    a = jnp.concatenate(
        [jnp.maximum(dis * s_ref[...] + b[0:1, FC * q:FC * (q + 1)], 0.0)
         for q, s_ref in enumerate((s0_ref, s1_ref, s2_ref, s3_ref))],
        axis=1)
    h = jnp.dot(a, w_ref[...], preferred_element_type=jnp.float32)
    hp_ref[...] = h * dis


def _fin_body(s0_ref, s1_ref, p_ref, b_ref, out_ref):
    dis = _dis_block(p_ref)
    o0 = dis * s0_ref[...] + b_ref[0:1, 0:FC]
    o1 = dis * s1_ref[...] + b_ref[0:1, FC:2 * FC]
    out_ref[...] = jnp.concatenate([o0, o1], axis=1)


def _mm1_call(x, W1s, pT):
    return pl.pallas_call(
        _mm1_body,
        grid=(NR, 4),
        in_specs=[
            pl.BlockSpec((BR, 256), lambda i, q: (i, 0)),
            pl.BlockSpec((256, FC), lambda i, q: (q, 0)),
            pl.BlockSpec((BR, 16), lambda i, q: (i, 0)),
        ],
        out_specs=pl.BlockSpec((BR, FC), lambda i, q: (q * NR + i, 0)),
        out_shape=jax.ShapeDtypeStruct((4 * N, FC), jnp.float32),
    )(x, W1s, pT)


def _mid_call(S, pT, b, Ws, nch_out):
    return pl.pallas_call(
        _mid_body,
        grid=(NR, nch_out),
        in_specs=[
            pl.BlockSpec((BR, FC), lambda i, q: (i, 0)),
            pl.BlockSpec((BR, FC), lambda i, q: (NR + i, 0)),
            pl.BlockSpec((BR, FC), lambda i, q: (2 * NR + i, 0)),
            pl.BlockSpec((BR, FC), lambda i, q: (3 * NR + i, 0)),
            pl.BlockSpec((BR, 16), lambda i, q: (i, 0)),
            pl.BlockSpec((1, 256), lambda i, q: (0, 0)),
            pl.BlockSpec((256, FC), lambda i, q: (q, 0)),
        ],
        out_specs=pl.BlockSpec((BR, FC), lambda i, q: (q * NR + i, 0)),
        out_shape=jax.ShapeDtypeStruct((nch_out * N, FC), jnp.float32),
    )(S, S, S, S, pT, b, Ws)


def _fin_call(S, pT, b):
    return pl.pallas_call(
        _fin_body,
        grid=(NR,),
        in_specs=[
            pl.BlockSpec((BR, FC), lambda i: (i, 0)),
            pl.BlockSpec((BR, FC), lambda i: (NR + i, 0)),
            pl.BlockSpec((BR, 16), lambda i: (i, 0)),
            pl.BlockSpec((1, 128), lambda i: (0, 0)),
        ],
        out_specs=pl.BlockSpec((BR, 128), lambda i: (i, 0)),
        out_shape=jax.ShapeDtypeStruct((N, 128), jnp.float32),
    )(S, S, pT, b)


def _stack_w(W):
    return jnp.concatenate(
        [W[:, q * FC:(q + 1) * FC] for q in range(W.shape[1] // FC)], axis=0)


def kernel(x, edge_index, W1, b1, W2, b2, W3, b3):
    src = edge_index[0].astype(jnp.int32)
    dst = edge_index[1].astype(jnp.int32)

    npad_agg = NP - E // NS
    pad_src = (jnp.arange(npad_agg, dtype=jnp.int32) * 41) % N
    pad_dst = N + jnp.arange(npad_agg, dtype=jnp.int32) % (NP - N)
    srcp = jnp.concatenate(
        [src.reshape(NS, E // NS),
         jnp.broadcast_to(pad_src, (NS, npad_agg))], axis=1)
    dstp = jnp.concatenate(
        [dst.reshape(NS, E // NS),
         jnp.broadcast_to(pad_dst, (NS, npad_agg))], axis=1)
    srcq = jnp.concatenate([srcp + q * N for q in range(4)], axis=0)
    srcq = srcq.reshape(4 * NS * AGG_ROWS, 128)
    dst2 = dstp.reshape(NS * AGG_ROWS, 128)

    zer = jnp.zeros((1024, 8), jnp.float32)
    one = jnp.ones((128, 8), jnp.float32)

    pT = _deg_kernel()(dst2, zer, one)

    hp1 = _mm1_call(x, _stack_w(W1), pT)
    S1 = _agg_kernel(4)(hp1, srcq, dst2)
    hp2 = _mid_call(S1, pT, b1.reshape(1, 256), _stack_w(W2), 4)
    S2 = _agg_kernel(4)(hp2, srcq, dst2)
    hp3 = _mid_call(S2, pT, b2.reshape(1, 256), _stack_w(W3), 2)
    S3 = _agg_kernel(2)(hp3, srcq, dst2)
    return _fin_call(S3, pT, b3.reshape(1, 128))


# speedup vs baseline: 1.0001x; 1.0001x over previous
import functools

import jax
import jax.numpy as jnp
from jax import lax
from jax.experimental import pallas as pl
from jax.experimental.pallas import tpu as pltpu
from jax.experimental.pallas import tpu_sc as plsc

N = 10000
E = 160000
NC = 2
NS = 16
NP = 10240
FC = 64

AGG_ROWS = 80
DEG_ROWS = 40

BR = 2000
NR = N // BR


@functools.cache
def _mesh():
    return plsc.VectorSubcoreMesh(
        core_axis_name="c", subcore_axis_name="s", num_cores=NC, num_subcores=NS
    )


def _deg_body(dst_ref, zer_ref, one_ref, out_ref, ones_v, idx_v, acc_s):
    c = lax.axis_index("c")
    s = lax.axis_index("s")
    @pl.when(s < 10)
    def _():
        pltpu.sync_copy(zer_ref, acc_s.at[pl.ds(s * 1024, 1024)])

    pltpu.sync_copy(one_ref, ones_v)
    pltpu.sync_copy(dst_ref.at[pl.ds(s * AGG_ROWS + c * DEG_ROWS, DEG_ROWS)],
                    idx_v)
    plsc.subcore_barrier()

    def batch(j, carry):
        pltpu.sync_copy(ones_v, acc_s.at[idx_v.at[j]], add=True)
        return carry

    lax.fori_loop(0, DEG_ROWS, batch, 0)
    plsc.subcore_barrier()

    @pl.when(s < 10)
    def _():
        r0 = s * 1024
        pltpu.sync_copy(acc_s.at[pl.ds(r0, 1024)],
                        out_ref.at[pl.ds(r0, 1024), pl.ds(c * 8, 8)])


@functools.cache
def _deg_kernel():
    return pl.kernel(
        _deg_body,
        out_type=jax.ShapeDtypeStruct((NP, 2 * 8), jnp.float32),
        mesh=_mesh(),
        compiler_params=pltpu.CompilerParams(use_tc_tiling_on_sc=False),
        scratch_types=[
            pltpu.VMEM((128, 8), jnp.float32),
            pltpu.VMEM((DEG_ROWS, 128), jnp.int32),
            pltpu.VMEM_SHARED((NP, 8), jnp.float32),
        ],
    )


def _agg_body(hp_ref, srcq_ref, dst_ref, out_ref,
              sidx_v, didx_v, rows_vs, gsems, ssems, acc_s, *, ph):
    c = lax.axis_index("c")
    s = lax.axis_index("s")
    nb = len(rows_vs)
    pltpu.sync_copy(dst_ref.at[pl.ds(s * AGG_ROWS, AGG_ROWS)], didx_v)

    for p in range(ph):
        q = 2 * p + c
        pltpu.sync_copy(srcq_ref.at[pl.ds((q * NS + s) * AGG_ROWS, AGG_ROWS)],
                        sidx_v)
        @pl.when(s < 10)
        def _():
            pltpu.sync_copy(hp_ref.at[pl.ds(q * N + s * 1000, 1000)],
                            acc_s.at[pl.ds(s * 1000, 1000)])

        plsc.subcore_barrier()

        def gather(m, b):
            return pltpu.make_async_copy(hp_ref.at[sidx_v.at[m]],
                                         rows_vs[b], gsems[b])

        def scatter(m, b):
            return pltpu.make_async_copy(rows_vs[b], acc_s.at[didx_v.at[m]],
                                         ssems[b])

        G = 6
        for b in range(G):
            gather(b, b).start()

        def batch(j, carry):
            m0 = nb * j
            for u in range(nb):
                m = m0 + u
                gather(m, u).wait()
                scatter(m, u).start(add=True)
                k = m + G
                bk = (u + G) % nb

                @pl.when(k < AGG_ROWS)
                def _():
                    @pl.when(k >= nb)
                    def _():
                        scatter(k - nb, bk).wait()

                    gather(k, bk).start()

            return carry

        lax.fori_loop(0, AGG_ROWS // nb, batch, 0)
        for u in range(nb):
            scatter(AGG_ROWS - nb + u, (AGG_ROWS - nb + u) % nb).wait()
        plsc.subcore_barrier()

        @pl.when(s < 10)
        def _():
            pltpu.sync_copy(acc_s.at[pl.ds(s * 1000, 1000)],
                            out_ref.at[pl.ds(q * N + s * 1000, 1000)])

        plsc.subcore_barrier()


@functools.cache
def _agg_kernel(nch):
    ph = nch // NC
    return pl.kernel(
        functools.partial(_agg_body, ph=ph),
        out_type=jax.ShapeDtypeStruct((nch * N, FC), jnp.float32),
        mesh=_mesh(),
        compiler_params=pltpu.CompilerParams(use_tc_tiling_on_sc=False),
        scratch_types=[
            pltpu.VMEM((AGG_ROWS, 128), jnp.int32),
            pltpu.VMEM((AGG_ROWS, 128), jnp.int32),
            tuple(pltpu.VMEM((128, FC), jnp.float32) for _ in range(8)),
            tuple(pltpu.SemaphoreType.DMA for _ in range(8)),
            tuple(pltpu.SemaphoreType.DMA for _ in range(8)),
            pltpu.VMEM_SHARED((NP, FC), jnp.float32),
        ],
    )


def _dis_block(p_ref):
    p = p_ref[...]
    return lax.rsqrt(p[:, 0:1] + p[:, 8:9] + 1.0)


def _mm1_body(x_ref, w_ref, p_ref, hp_ref):
    dis = _dis_block(p_ref)
    h = jnp.dot(x_ref[...], w_ref[...], preferred_element_type=jnp.float32)
    hp_ref[...] = h * dis


def _mid_body(s0_ref, s1_ref, s2_ref, s3_ref, p_ref, b_ref, w_ref, hp_ref):
    dis = _dis_block(p_ref)
    b = b_ref[...]
    a = jnp.concatenate(
        [jnp.maximum(dis * s_ref[...] + b[0:1, FC * q:FC * (q + 1)], 0.0)
         for q, s_ref in enumerate((s0_ref, s1_ref, s2_ref, s3_ref))],
        axis=1)
    h = jnp.dot(a, w_ref[...], preferred_element_type=jnp.float32)
    hp_ref[...] = h * dis


def _fin_body(s0_ref, s1_ref, p_ref, b_ref, out_ref):
    dis = _dis_block(p_ref)
    o0 = dis * s0_ref[...] + b_ref[0:1, 0:FC]
    o1 = dis * s1_ref[...] + b_ref[0:1, FC:2 * FC]
    out_ref[...] = jnp.concatenate([o0, o1], axis=1)


def _mm1_call(x, W1s, pT):
    return pl.pallas_call(
        _mm1_body,
        grid=(NR, 4),
        in_specs=[
            pl.BlockSpec((BR, 256), lambda i, q: (i, 0)),
            pl.BlockSpec((256, FC), lambda i, q: (q, 0)),
            pl.BlockSpec((BR, 16), lambda i, q: (i, 0)),
        ],
        out_specs=pl.BlockSpec((BR, FC), lambda i, q: (q * NR + i, 0)),
        out_shape=jax.ShapeDtypeStruct((4 * N, FC), jnp.float32),
    )(x, W1s, pT)


def _mid_call(S, pT, b, Ws, nch_out):
    return pl.pallas_call(
        _mid_body,
        grid=(NR, nch_out),
        in_specs=[
            pl.BlockSpec((BR, FC), lambda i, q: (i, 0)),
            pl.BlockSpec((BR, FC), lambda i, q: (NR + i, 0)),
            pl.BlockSpec((BR, FC), lambda i, q: (2 * NR + i, 0)),
            pl.BlockSpec((BR, FC), lambda i, q: (3 * NR + i, 0)),
            pl.BlockSpec((BR, 16), lambda i, q: (i, 0)),
            pl.BlockSpec((1, 256), lambda i, q: (0, 0)),
            pl.BlockSpec((256, FC), lambda i, q: (q, 0)),
        ],
        out_specs=pl.BlockSpec((BR, FC), lambda i, q: (q * NR + i, 0)),
        out_shape=jax.ShapeDtypeStruct((nch_out * N, FC), jnp.float32),
    )(S, S, S, S, pT, b, Ws)


def _fin_call(S, pT, b):
    return pl.pallas_call(
        _fin_body,
        grid=(NR,),
        in_specs=[
            pl.BlockSpec((BR, FC), lambda i: (i, 0)),
            pl.BlockSpec((BR, FC), lambda i: (NR + i, 0)),
            pl.BlockSpec((BR, 16), lambda i: (i, 0)),
            pl.BlockSpec((1, 128), lambda i: (0, 0)),
        ],
        out_specs=pl.BlockSpec((BR, 128), lambda i: (i, 0)),
        out_shape=jax.ShapeDtypeStruct((N, 128), jnp.float32),
    )(S, S, pT, b)


def _stack_w(W):
    return jnp.concatenate(
        [W[:, q * FC:(q + 1) * FC] for q in range(W.shape[1] // FC)], axis=0)


def kernel(x, edge_index, W1, b1, W2, b2, W3, b3):
    src = edge_index[0].astype(jnp.int32)
    dst = edge_index[1].astype(jnp.int32)

    npad_agg = NP - E // NS
    pad_src = (jnp.arange(npad_agg, dtype=jnp.int32) * 41) % N
    pad_dst = N + jnp.arange(npad_agg, dtype=jnp.int32) % (NP - N)
    srcp = jnp.concatenate(
        [src.reshape(NS, E // NS),
         jnp.broadcast_to(pad_src, (NS, npad_agg))], axis=1)
    dstp = jnp.concatenate(
        [dst.reshape(NS, E // NS),
         jnp.broadcast_to(pad_dst, (NS, npad_agg))], axis=1)
    srcq = jnp.concatenate([srcp + q * N for q in range(4)], axis=0)
    srcq = srcq.reshape(4 * NS * AGG_ROWS, 128)
    dst2 = dstp.reshape(NS * AGG_ROWS, 128)

    zer = jnp.zeros((1024, 8), jnp.float32)
    one = jnp.ones((128, 8), jnp.float32)

    pT = _deg_kernel()(dst2, zer, one)

    hp1 = _mm1_call(x, _stack_w(W1), pT)
    S1 = _agg_kernel(4)(hp1, srcq, dst2)
    hp2 = _mid_call(S1, pT, b1.reshape(1, 256), _stack_w(W2), 4)
    S2 = _agg_kernel(4)(hp2, srcq, dst2)
    hp3 = _mid_call(S2, pT, b2.reshape(1, 256), _stack_w(W3), 2)
    S3 = _agg_kernel(2)(hp3, srcq, dst2)
    return _fin_call(S3, pT, b3.reshape(1, 128))


# speedup vs baseline: 1.0281x; 1.0280x over previous
import functools

import jax
import jax.numpy as jnp
from jax import lax
from jax.experimental import pallas as pl
from jax.experimental.pallas import tpu as pltpu
from jax.experimental.pallas import tpu_sc as plsc

N = 10000
E = 160000
NC = 2
NS = 16
NP = 10240
FC = 64

AGG_ROWS = 80
DEG_ROWS = 40

BR = 10000
NR = N // BR


@functools.cache
def _mesh():
    return plsc.VectorSubcoreMesh(
        core_axis_name="c", subcore_axis_name="s", num_cores=NC, num_subcores=NS
    )


def _deg_body(dst_ref, zer_ref, one_ref, out_ref, ones_v, idx_v, acc_s):
    c = lax.axis_index("c")
    s = lax.axis_index("s")
    @pl.when(s < 10)
    def _():
        pltpu.sync_copy(zer_ref, acc_s.at[pl.ds(s * 1024, 1024)])

    pltpu.sync_copy(one_ref, ones_v)
    pltpu.sync_copy(dst_ref.at[pl.ds(s * AGG_ROWS + c * DEG_ROWS, DEG_ROWS)],
                    idx_v)
    plsc.subcore_barrier()

    def batch(j, carry):
        pltpu.sync_copy(ones_v, acc_s.at[idx_v.at[j]], add=True)
        return carry

    lax.fori_loop(0, DEG_ROWS, batch, 0)
    plsc.subcore_barrier()

    @pl.when(s < 10)
    def _():
        r0 = s * 1024
        pltpu.sync_copy(acc_s.at[pl.ds(r0, 1024)],
                        out_ref.at[pl.ds(r0, 1024), pl.ds(c * 8, 8)])


@functools.cache
def _deg_kernel():
    return pl.kernel(
        _deg_body,
        out_type=jax.ShapeDtypeStruct((NP, 2 * 8), jnp.float32),
        mesh=_mesh(),
        compiler_params=pltpu.CompilerParams(use_tc_tiling_on_sc=False),
        scratch_types=[
            pltpu.VMEM((128, 8), jnp.float32),
            pltpu.VMEM((DEG_ROWS, 128), jnp.int32),
            pltpu.VMEM_SHARED((NP, 8), jnp.float32),
        ],
    )


def _agg_body(hp_ref, srcq_ref, dst_ref, out_ref,
              sidx_v, didx_v, rows_vs, gsems, ssems, acc_s, *, ph):
    c = lax.axis_index("c")
    s = lax.axis_index("s")
    nb = len(rows_vs)
    pltpu.sync_copy(dst_ref.at[pl.ds(s * AGG_ROWS, AGG_ROWS)], didx_v)

    for p in range(ph):
        q = 2 * p + c
        pltpu.sync_copy(srcq_ref.at[pl.ds((q * NS + s) * AGG_ROWS, AGG_ROWS)],
                        sidx_v)
        @pl.when(s < 10)
        def _():
            pltpu.sync_copy(hp_ref.at[pl.ds(q * N + s * 1000, 1000)],
                            acc_s.at[pl.ds(s * 1000, 1000)])

        plsc.subcore_barrier()

        def gather(m, b):
            return pltpu.make_async_copy(hp_ref.at[sidx_v.at[m]],
                                         rows_vs[b], gsems[b])

        def scatter(m, b):
            return pltpu.make_async_copy(rows_vs[b], acc_s.at[didx_v.at[m]],
                                         ssems[b])

        G = 6
        for b in range(G):
            gather(b, b).start()

        def batch(j, carry):
            m0 = nb * j
            for u in range(nb):
                m = m0 + u
                gather(m, u).wait()
                scatter(m, u).start(add=True)
                k = m + G
                bk = (u + G) % nb

                @pl.when(k < AGG_ROWS)
                def _():
                    @pl.when(k >= nb)
                    def _():
                        scatter(k - nb, bk).wait()

                    gather(k, bk).start()

            return carry

        lax.fori_loop(0, AGG_ROWS // nb, batch, 0)
        for u in range(nb):
            scatter(AGG_ROWS - nb + u, (AGG_ROWS - nb + u) % nb).wait()
        plsc.subcore_barrier()

        @pl.when(s < 10)
        def _():
            pltpu.sync_copy(acc_s.at[pl.ds(s * 1000, 1000)],
                            out_ref.at[pl.ds(q * N + s * 1000, 1000)])

        plsc.subcore_barrier()


@functools.cache
def _agg_kernel(nch):
    ph = nch // NC
    return pl.kernel(
        functools.partial(_agg_body, ph=ph),
        out_type=jax.ShapeDtypeStruct((nch * N, FC), jnp.float32),
        mesh=_mesh(),
        compiler_params=pltpu.CompilerParams(use_tc_tiling_on_sc=False),
        scratch_types=[
            pltpu.VMEM((AGG_ROWS, 128), jnp.int32),
            pltpu.VMEM((AGG_ROWS, 128), jnp.int32),
            tuple(pltpu.VMEM((128, FC), jnp.float32) for _ in range(8)),
            tuple(pltpu.SemaphoreType.DMA for _ in range(8)),
            tuple(pltpu.SemaphoreType.DMA for _ in range(8)),
            pltpu.VMEM_SHARED((NP, FC), jnp.float32),
        ],
    )


def _dis_block(p_ref):
    p = p_ref[...]
    return lax.rsqrt(p[:, 0:1] + p[:, 8:9] + 1.0)


def _mm1_body(x_ref, w_ref, p_ref, hp_ref):
    dis = _dis_block(p_ref)
    h = jnp.dot(x_ref[...], w_ref[...], preferred_element_type=jnp.float32)
    hp_ref[...] = h * dis


def _mid_body(s0_ref, s1_ref, s2_ref, s3_ref, p_ref, b_ref, w_ref, hp_ref):
    dis = _dis_block(p_ref)
    b = b_ref[...]
    a = jnp.concatenate(
        [jnp.maximum(dis * s_ref[...] + b[0:1, FC * q:FC * (q + 1)], 0.0)
         for q, s_ref in enumerate((s0_ref, s1_ref, s2_ref, s3_ref))],
        axis=1)
    h = jnp.dot(a, w_ref[...], preferred_element_type=jnp.float32)
    hp_ref[...] = h * dis


def _fin_body(s0_ref, s1_ref, p_ref, b_ref, out_ref):
    dis = _dis_block(p_ref)
    o0 = dis * s0_ref[...] + b_ref[0:1, 0:FC]
    o1 = dis * s1_ref[...] + b_ref[0:1, FC:2 * FC]
    out_ref[...] = jnp.concatenate([o0, o1], axis=1)


def _mm1_call(x, W1s, pT):
    return pl.pallas_call(
        _mm1_body,
        grid=(NR, 4),
        in_specs=[
            pl.BlockSpec((BR, 256), lambda i, q: (i, 0)),
            pl.BlockSpec((256, FC), lambda i, q: (q, 0)),
            pl.BlockSpec((BR, 16), lambda i, q: (i, 0)),
        ],
        out_specs=pl.BlockSpec((BR, FC), lambda i, q: (q * NR + i, 0)),
        out_shape=jax.ShapeDtypeStruct((4 * N, FC), jnp.float32),
    )(x, W1s, pT)


def _mid_call(S, pT, b, Ws, nch_out):
    return pl.pallas_call(
        _mid_body,
        grid=(NR, nch_out),
        in_specs=[
            pl.BlockSpec((BR, FC), lambda i, q: (i, 0)),
            pl.BlockSpec((BR, FC), lambda i, q: (NR + i, 0)),
            pl.BlockSpec((BR, FC), lambda i, q: (2 * NR + i, 0)),
            pl.BlockSpec((BR, FC), lambda i, q: (3 * NR + i, 0)),
            pl.BlockSpec((BR, 16), lambda i, q: (i, 0)),
            pl.BlockSpec((1, 256), lambda i, q: (0, 0)),
            pl.BlockSpec((256, FC), lambda i, q: (q, 0)),
        ],
        out_specs=pl.BlockSpec((BR, FC), lambda i, q: (q * NR + i, 0)),
        out_shape=jax.ShapeDtypeStruct((nch_out * N, FC), jnp.float32),
    )(S, S, S, S, pT, b, Ws)


def _fin_call(S, pT, b):
    return pl.pallas_call(
        _fin_body,
        grid=(NR,),
        in_specs=[
            pl.BlockSpec((BR, FC), lambda i: (i, 0)),
            pl.BlockSpec((BR, FC), lambda i: (NR + i, 0)),
            pl.BlockSpec((BR, 16), lambda i: (i, 0)),
            pl.BlockSpec((1, 128), lambda i: (0, 0)),
        ],
        out_specs=pl.BlockSpec((BR, 128), lambda i: (i, 0)),
        out_shape=jax.ShapeDtypeStruct((N, 128), jnp.float32),
    )(S, S, pT, b)


def _stack_w(W):
    return jnp.concatenate(
        [W[:, q * FC:(q + 1) * FC] for q in range(W.shape[1] // FC)], axis=0)


def kernel(x, edge_index, W1, b1, W2, b2, W3, b3):
    src = edge_index[0].astype(jnp.int32)
    dst = edge_index[1].astype(jnp.int32)

    npad_agg = NP - E // NS
    pad_src = (jnp.arange(npad_agg, dtype=jnp.int32) * 41) % N
    pad_dst = N + jnp.arange(npad_agg, dtype=jnp.int32) % (NP - N)
    srcp = jnp.concatenate(
        [src.reshape(NS, E // NS),
         jnp.broadcast_to(pad_src, (NS, npad_agg))], axis=1)
    dstp = jnp.concatenate(
        [dst.reshape(NS, E // NS),
         jnp.broadcast_to(pad_dst, (NS, npad_agg))], axis=1)
    srcq = jnp.concatenate([srcp + q * N for q in range(4)], axis=0)
    srcq = srcq.reshape(4 * NS * AGG_ROWS, 128)
    dst2 = dstp.reshape(NS * AGG_ROWS, 128)

    zer = jnp.zeros((1024, 8), jnp.float32)
    one = jnp.ones((128, 8), jnp.float32)

    pT = _deg_kernel()(dst2, zer, one)

    hp1 = _mm1_call(x, _stack_w(W1), pT)
    S1 = _agg_kernel(4)(hp1, srcq, dst2)
    hp2 = _mid_call(S1, pT, b1.reshape(1, 256), _stack_w(W2), 4)
    S2 = _agg_kernel(4)(hp2, srcq, dst2)
    hp3 = _mid_call(S2, pT, b2.reshape(1, 256), _stack_w(W3), 2)
    S3 = _agg_kernel(2)(hp3, srcq, dst2)
    return _fin_call(S3, pT, b3.reshape(1, 128))


# speedup vs baseline: 1.0325x; 1.0042x over previous
import functools

import jax
import jax.numpy as jnp
from jax import lax
from jax.experimental import pallas as pl
from jax.experimental.pallas import tpu as pltpu
from jax.experimental.pallas import tpu_sc as plsc

N = 10000
E = 160000
NC = 2
NS = 16
NP = 10240
FC = 64

AGG_ROWS = 80
DEG_ROWS = 40

BR = 10000
NR = N // BR


@functools.cache
def _mesh():
    return plsc.VectorSubcoreMesh(
        core_axis_name="c", subcore_axis_name="s", num_cores=NC, num_subcores=NS
    )


def _deg_body(dst_ref, zer_ref, one_ref, out_ref, ones_v, idx_v, dsem, acc_s):
    c = lax.axis_index("c")
    s = lax.axis_index("s")
    @pl.when(s < 10)
    def _():
        pltpu.sync_copy(zer_ref, acc_s.at[pl.ds(s * 1024, 1024)])

    pltpu.sync_copy(one_ref, ones_v)
    pltpu.sync_copy(dst_ref.at[pl.ds(s * AGG_ROWS + c * DEG_ROWS, DEG_ROWS)],
                    idx_v)
    plsc.subcore_barrier()

    def batch(j, carry):
        pltpu.make_async_copy(ones_v, acc_s.at[idx_v.at[j]],
                              dsem).start(add=True)
        return carry

    lax.fori_loop(0, DEG_ROWS, batch, 0)

    def drain(j, carry):
        pltpu.make_async_copy(ones_v, acc_s.at[idx_v.at[j]], dsem).wait()
        return carry

    lax.fori_loop(0, DEG_ROWS, drain, 0)
    plsc.subcore_barrier()

    @pl.when(s < 10)
    def _():
        r0 = s * 1024
        pltpu.sync_copy(acc_s.at[pl.ds(r0, 1024)],
                        out_ref.at[pl.ds(r0, 1024), pl.ds(c * 8, 8)])


@functools.cache
def _deg_kernel():
    return pl.kernel(
        _deg_body,
        out_type=jax.ShapeDtypeStruct((NP, 2 * 8), jnp.float32),
        mesh=_mesh(),
        compiler_params=pltpu.CompilerParams(use_tc_tiling_on_sc=False),
        scratch_types=[
            pltpu.VMEM((128, 8), jnp.float32),
            pltpu.VMEM((DEG_ROWS, 128), jnp.int32),
            pltpu.SemaphoreType.DMA,
            pltpu.VMEM_SHARED((NP, 8), jnp.float32),
        ],
    )


def _agg_body(hp_ref, srcq_ref, dst_ref, out_ref,
              sidx_v, didx_v, rows_vs, gsems, ssems, acc_s, *, ph):
    c = lax.axis_index("c")
    s = lax.axis_index("s")
    nb = len(rows_vs)
    pltpu.sync_copy(dst_ref.at[pl.ds(s * AGG_ROWS, AGG_ROWS)], didx_v)

    for p in range(ph):
        q = 2 * p + c
        pltpu.sync_copy(srcq_ref.at[pl.ds((q * NS + s) * AGG_ROWS, AGG_ROWS)],
                        sidx_v)
        @pl.when(s < 10)
        def _():
            pltpu.sync_copy(hp_ref.at[pl.ds(q * N + s * 1000, 1000)],
                            acc_s.at[pl.ds(s * 1000, 1000)])

        plsc.subcore_barrier()

        def gather(m, b):
            return pltpu.make_async_copy(hp_ref.at[sidx_v.at[m]],
                                         rows_vs[b], gsems[b])

        def scatter(m, b):
            return pltpu.make_async_copy(rows_vs[b], acc_s.at[didx_v.at[m]],
                                         ssems[b])

        G = 6
        for b in range(G):
            gather(b, b).start()

        def batch(j, carry):
            m0 = nb * j
            for u in range(nb):
                m = m0 + u
                gather(m, u).wait()
                scatter(m, u).start(add=True)
                k = m + G
                bk = (u + G) % nb

                @pl.when(k < AGG_ROWS)
                def _():
                    @pl.when(k >= nb)
                    def _():
                        scatter(k - nb, bk).wait()

                    gather(k, bk).start()

            return carry

        lax.fori_loop(0, AGG_ROWS // nb, batch, 0)
        for u in range(nb):
            scatter(AGG_ROWS - nb + u, (AGG_ROWS - nb + u) % nb).wait()
        plsc.subcore_barrier()

        @pl.when(s < 10)
        def _():
            pltpu.sync_copy(acc_s.at[pl.ds(s * 1000, 1000)],
                            out_ref.at[pl.ds(q * N + s * 1000, 1000)])

        plsc.subcore_barrier()


@functools.cache
def _agg_kernel(nch):
    ph = nch // NC
    return pl.kernel(
        functools.partial(_agg_body, ph=ph),
        out_type=jax.ShapeDtypeStruct((nch * N, FC), jnp.float32),
        mesh=_mesh(),
        compiler_params=pltpu.CompilerParams(use_tc_tiling_on_sc=False),
        scratch_types=[
            pltpu.VMEM((AGG_ROWS, 128), jnp.int32),
            pltpu.VMEM((AGG_ROWS, 128), jnp.int32),
            tuple(pltpu.VMEM((128, FC), jnp.float32) for _ in range(8)),
            tuple(pltpu.SemaphoreType.DMA for _ in range(8)),
            tuple(pltpu.SemaphoreType.DMA for _ in range(8)),
            pltpu.VMEM_SHARED((NP, FC), jnp.float32),
        ],
    )


def _dis_block(p_ref):
    p = p_ref[...]
    return lax.rsqrt(p[:, 0:1] + p[:, 8:9] + 1.0)


def _mm1_body(x_ref, w_ref, p_ref, hp_ref):
    dis = _dis_block(p_ref)
    h = jnp.dot(x_ref[...], w_ref[...], preferred_element_type=jnp.float32)
    hp_ref[...] = h * dis


def _mid_body(s0_ref, s1_ref, s2_ref, s3_ref, p_ref, b_ref, w_ref, hp_ref):
    dis = _dis_block(p_ref)
    b = b_ref[...]
    a = jnp.concatenate(
        [jnp.maximum(dis * s_ref[...] + b[0:1, FC * q:FC * (q + 1)], 0.0)
         for q, s_ref in enumerate((s0_ref, s1_ref, s2_ref, s3_ref))],
        axis=1)
    h = jnp.dot(a, w_ref[...], preferred_element_type=jnp.float32)
    hp_ref[...] = h * dis


def _fin_body(s0_ref, s1_ref, p_ref, b_ref, out_ref):
    dis = _dis_block(p_ref)
    o0 = dis * s0_ref[...] + b_ref[0:1, 0:FC]
    o1 = dis * s1_ref[...] + b_ref[0:1, FC:2 * FC]
    out_ref[...] = jnp.concatenate([o0, o1], axis=1)


def _mm1_call(x, W1s, pT):
    return pl.pallas_call(
        _mm1_body,
        grid=(NR, 4),
        in_specs=[
            pl.BlockSpec((BR, 256), lambda i, q: (i, 0)),
            pl.BlockSpec((256, FC), lambda i, q: (q, 0)),
            pl.BlockSpec((BR, 16), lambda i, q: (i, 0)),
        ],
        out_specs=pl.BlockSpec((BR, FC), lambda i, q: (q * NR + i, 0)),
        out_shape=jax.ShapeDtypeStruct((4 * N, FC), jnp.float32),
    )(x, W1s, pT)


def _mid_call(S, pT, b, Ws, nch_out):
    return pl.pallas_call(
        _mid_body,
        grid=(NR, nch_out),
        in_specs=[
            pl.BlockSpec((BR, FC), lambda i, q: (i, 0)),
            pl.BlockSpec((BR, FC), lambda i, q: (NR + i, 0)),
            pl.BlockSpec((BR, FC), lambda i, q: (2 * NR + i, 0)),
            pl.BlockSpec((BR, FC), lambda i, q: (3 * NR + i, 0)),
            pl.BlockSpec((BR, 16), lambda i, q: (i, 0)),
            pl.BlockSpec((1, 256), lambda i, q: (0, 0)),
            pl.BlockSpec((256, FC), lambda i, q: (q, 0)),
        ],
        out_specs=pl.BlockSpec((BR, FC), lambda i, q: (q * NR + i, 0)),
        out_shape=jax.ShapeDtypeStruct((nch_out * N, FC), jnp.float32),
    )(S, S, S, S, pT, b, Ws)


def _fin_call(S, pT, b):
    return pl.pallas_call(
        _fin_body,
        grid=(NR,),
        in_specs=[
            pl.BlockSpec((BR, FC), lambda i: (i, 0)),
            pl.BlockSpec((BR, FC), lambda i: (NR + i, 0)),
            pl.BlockSpec((BR, 16), lambda i: (i, 0)),
            pl.BlockSpec((1, 128), lambda i: (0, 0)),
        ],
        out_specs=pl.BlockSpec((BR, 128), lambda i: (i, 0)),
        out_shape=jax.ShapeDtypeStruct((N, 128), jnp.float32),
    )(S, S, pT, b)


def _stack_w(W):
    return jnp.concatenate(
        [W[:, q * FC:(q + 1) * FC] for q in range(W.shape[1] // FC)], axis=0)


def kernel(x, edge_index, W1, b1, W2, b2, W3, b3):
    src = edge_index[0].astype(jnp.int32)
    dst = edge_index[1].astype(jnp.int32)

    npad_agg = NP - E // NS
    pad_src = (jnp.arange(npad_agg, dtype=jnp.int32) * 41) % N
    pad_dst = N + jnp.arange(npad_agg, dtype=jnp.int32) % (NP - N)
    srcp = jnp.concatenate(
        [src.reshape(NS, E // NS),
         jnp.broadcast_to(pad_src, (NS, npad_agg))], axis=1)
    dstp = jnp.concatenate(
        [dst.reshape(NS, E // NS),
         jnp.broadcast_to(pad_dst, (NS, npad_agg))], axis=1)
    srcq = jnp.concatenate([srcp + q * N for q in range(4)], axis=0)
    srcq = srcq.reshape(4 * NS * AGG_ROWS, 128)
    dst2 = dstp.reshape(NS * AGG_ROWS, 128)

    zer = jnp.zeros((1024, 8), jnp.float32)
    one = jnp.ones((128, 8), jnp.float32)

    pT = _deg_kernel()(dst2, zer, one)

    hp1 = _mm1_call(x, _stack_w(W1), pT)
    S1 = _agg_kernel(4)(hp1, srcq, dst2)
    hp2 = _mid_call(S1, pT, b1.reshape(1, 256), _stack_w(W2), 4)
    S2 = _agg_kernel(4)(hp2, srcq, dst2)
    hp3 = _mid_call(S2, pT, b2.reshape(1, 256), _stack_w(W3), 2)
    S3 = _agg_kernel(2)(hp3, srcq, dst2)
    return _fin_call(S3, pT, b3.reshape(1, 128))
